# baseline (speedup 1.0000x reference)
import functools

import jax
import jax.numpy as jnp
from jax import lax
from jax.experimental import pallas as pl
from jax.experimental.pallas import tpu as pltpu
from jax.experimental.pallas import tpu_sc as plsc

_DIM = 64
_B = 4096
_NC, _NS, _L = 2, 16, 16
_NW = _NC * _NS
_BPW = _B // _NW
_NU = 100000


@functools.cache
def _make_gather_sc():
  mesh = plsc.VectorSubcoreMesh(
      core_axis_name="c", subcore_axis_name="s",
      num_cores=_NC, num_subcores=_NS)

  @functools.partial(
      pl.kernel,
      mesh=mesh,
      compiler_params=pltpu.CompilerParams(use_tc_tiling_on_sc=True),
      out_type=(
          jax.ShapeDtypeStruct((_B, _DIM), jnp.float32),
          jax.ShapeDtypeStruct((_B, _DIM), jnp.float32),
          jax.ShapeDtypeStruct((_B, _DIM), jnp.float32),
          jax.ShapeDtypeStruct((_B, _DIM), jnp.float32),
          jax.ShapeDtypeStruct((_B,), jnp.float32),
          jax.ShapeDtypeStruct((_B,), jnp.float32),
      ),
      scratch_types=[
          pltpu.VMEM((_BPW,), jnp.int32),
          pltpu.VMEM((_BPW,), jnp.int32),
          pltpu.VMEM((_BPW,), jnp.int32),
          pltpu.VMEM((_BPW, _DIM), jnp.float32),
          pltpu.VMEM((_BPW, _DIM), jnp.float32),
          pltpu.VMEM((_BPW, _DIM), jnp.float32),
          pltpu.VMEM((_BPW, _DIM), jnp.float32),
          pltpu.VMEM((_BPW,), jnp.float32),
          pltpu.VMEM((_BPW,), jnp.float32),
          pltpu.SemaphoreType.DMA,
      ],
  )
  def _gather_sc(uid_hbm, iid_hbm, iidw_hbm, user_emb, item_emb, word_emb,
                 bias_u1, bias_i1,
                 ue_o, ug_o, ie_o, ig_o, bu_o, bi_o,
                 uid_v, iid_v, iidw_v, ue_v, ug_v, ie_v, ig_v, bu_v, bi_v, sem):
    wid = lax.axis_index("s") * _NC + lax.axis_index("c")
    base = wid * _BPW
    pltpu.sync_copy(uid_hbm.at[pl.ds(base, _BPW)], uid_v)
    pltpu.sync_copy(iid_hbm.at[pl.ds(base, _BPW)], iid_v)
    pltpu.sync_copy(iidw_hbm.at[pl.ds(base, _BPW)], iidw_v)

    def body(g, _):
      off = g * _L
      uvec = uid_v[pl.ds(off, _L)]
      ivec = iid_v[pl.ds(off, _L)]
      wvec = iidw_v[pl.ds(off, _L)]
      descs = []
      for l in range(_L):
        r = off + l
        descs.append(pltpu.async_copy(user_emb.at[uvec[l]], ue_v.at[r], sem))
        descs.append(pltpu.async_copy(word_emb.at[uvec[l]], ug_v.at[r], sem))
        descs.append(pltpu.async_copy(item_emb.at[ivec[l]], ie_v.at[r], sem))
        descs.append(pltpu.async_copy(word_emb.at[wvec[l]], ig_v.at[r], sem))
      for d in descs:
        d.wait()
      return 0

    lax.fori_loop(0, _BPW // _L, body, 0)
    pltpu.async_copy(bias_u1.at[uid_v], bu_v, sem).wait()
    pltpu.async_copy(bias_i1.at[iid_v], bi_v, sem).wait()
    pltpu.sync_copy(ue_v, ue_o.at[pl.ds(base, _BPW)])
    pltpu.sync_copy(ug_v, ug_o.at[pl.ds(base, _BPW)])
    pltpu.sync_copy(ie_v, ie_o.at[pl.ds(base, _BPW)])
    pltpu.sync_copy(ig_v, ig_o.at[pl.ds(base, _BPW)])
    pltpu.sync_copy(bu_v, bu_o.at[pl.ds(base, _BPW)])
    pltpu.sync_copy(bi_v, bi_o.at[pl.ds(base, _BPW)])

  return _gather_sc


def _fm_tc(ue, ug, ie, ig, v_ref, wrow_ref, bu, bi, bias_ref, out_ref):
  x = jnp.concatenate([ue[...], ug[...], ie[...], ig[...]], axis=1)
  v = v_ref[...]
  y = jnp.dot(x, v, preferred_element_type=jnp.float32)
  sv = jnp.sum(v * v, axis=1)
  lin = jnp.sum(x * wrow_ref[...], axis=1)
  t1 = jnp.sum(y * y, axis=1)
  t2 = jnp.sum((x * x) * sv[None, :], axis=1)
  out_ref[...] = lin + 0.5 * (t1 - t2) + bu[...] + bi[...] + bias_ref[0]


_fm_call = pl.pallas_call(
    _fm_tc,
    in_specs=[
        pl.BlockSpec(memory_space=pltpu.VMEM),
        pl.BlockSpec(memory_space=pltpu.VMEM),
        pl.BlockSpec(memory_space=pltpu.VMEM),
        pl.BlockSpec(memory_space=pltpu.VMEM),
        pl.BlockSpec(memory_space=pltpu.VMEM),
        pl.BlockSpec(memory_space=pltpu.VMEM),
        pl.BlockSpec(memory_space=pltpu.VMEM),
        pl.BlockSpec(memory_space=pltpu.VMEM),
        pl.BlockSpec(memory_space=pltpu.SMEM),
    ],
    out_shape=jax.ShapeDtypeStruct((_B,), jnp.float32),
)


def kernel(uid_batch, iid_batch, u_nodes, u_adj_ind, u_adj_tp,
           i_nodes, i_adj_ind, i_adj_tp,
           user_emb, item_emb, word_emb, W_lin, V, bias_u, bias_i, bias):
  uid = uid_batch.astype(jnp.int32)
  iid = iid_batch.astype(jnp.int32)
  ue, ug, ie, ig, bu, bi = _make_gather_sc()(
      uid, iid, iid + _NU, user_emb, item_emb, word_emb, bias_u, bias_i)
  return _fm_call(ue, ug, ie, ig, V, W_lin.reshape(1, -1), bu, bi, bias)


# speedup vs baseline: 1.6498x; 1.6498x over previous
import functools

import jax
import jax.numpy as jnp
from jax import lax
from jax.experimental import pallas as pl
from jax.experimental.pallas import tpu as pltpu
from jax.experimental.pallas import tpu_sc as plsc

_DIM = 64
_B = 4096
_NC, _NS, _L = 2, 16, 16
_NW = _NC * _NS
_NU = 100000

_CW = 10240
_NCHUNK = 10
_LASTW = 7808
_WBASE = 99968
_TAILW = 128

_TPK = 8
_BPT = _B // _TPK

_EDGES_LO = [c * _CW for c in range(_NCHUNK)] + [_WBASE, 1 << 30]
_EDGES_HI = [c * _CW - 32 for c in range(_NCHUNK)] + [99936, 1 << 30]


@functools.cache
def _make_gather_sc():
  mesh = plsc.VectorSubcoreMesh(
      core_axis_name="c", subcore_axis_name="s",
      num_cores=_NC, num_subcores=_NS)

  @functools.partial(
      pl.kernel,
      mesh=mesh,
      compiler_params=pltpu.CompilerParams(use_tc_tiling_on_sc=True, needs_layout_passes=False),
      out_type=(
          jax.ShapeDtypeStruct((4 * _DIM, _B), jnp.float32),
          jax.ShapeDtypeStruct((_B,), jnp.float32),
          jax.ShapeDtypeStruct((_B,), jnp.float32),
      ),
      scratch_types=[
          pltpu.VMEM((_B,), jnp.int32),
          pltpu.VMEM((_B,), jnp.int32),
          pltpu.VMEM((32,), jnp.int32),
          pltpu.VMEM((8, _CW), jnp.float32),
          pltpu.VMEM((8, _B), jnp.float32),
          pltpu.VMEM((_BPT,), jnp.int32),
          pltpu.VMEM((_BPT,), jnp.float32),
          pltpu.SemaphoreType.DMA,
      ],
  )
  def _gather_sc(uid_hbm, iid_hbm, us_hbm, uo_hbm, is_hbm, io_hbm, starts_hbm,
                 ueT, ieT, weT, tails, bias_u1, bias_i1,
                 xT_o, bu_o, bi_o,
                 idx_v, ord_v, starts_v, buf_v, rows_v, bidx_v, bseg_v, sem):
    wid = lax.axis_index("s") * _NC + lax.axis_index("c")
    kind = wid // _TPK
    d0 = pl.multiple_of((wid % _TPK) * 8, 8)
    lanes = jnp.arange(_L, dtype=jnp.int32)

    def scan_chunk(sv0, c, gb, ln):
      lo = sv0[c]
      hi = sv0[c + 1]
      zeros = jnp.zeros((_L,), dtype=jnp.int32)

      def urow(u, _, gb=gb, ln=ln, lo=lo, hi=hi):
        usplat = zeros + u

        def body(k, _):
          sl = pl.ds(k * _L, _L)
          pos = k * _L + lanes
          m = (pos >= lo) & (pos < hi)
          iv = idx_v[sl]
          ov = ord_v[sl]
          ic = jnp.clip(iv - gb, 0, ln - 1)
          vals = plsc.load_gather(buf_v, [usplat, ic])
          plsc.store_scatter(rows_v, [usplat, ov], vals, mask=m)
          return 0

        lax.fori_loop(lo // _L, (hi + _L - 1) // _L, body, 0)
        return 0

      lax.fori_loop(0, 8, urow, 0)

    def pick_rows(table, tab_base, idx_base, row_base, gb_tail):
      sv0 = starts_v[pl.ds(0, _L)]
      for c in range(_NCHUNK):
        ln = _CW if c < _NCHUNK - 1 else _LASTW
        n0 = tab_base + c * _CW
        pltpu.sync_copy(table.at[pl.ds(d0, 8), pl.ds(n0, ln)],
                        buf_v.at[:, pl.ds(0, ln)])
        scan_chunk(sv0, c, n0 - idx_base, ln)
      pltpu.sync_copy(tails.at[pl.ds(row_base + d0, 8)],
                      buf_v.at[:, pl.ds(0, _TAILW)])
      scan_chunk(sv0, _NCHUNK, gb_tail, _TAILW)
      pltpu.sync_copy(rows_v, xT_o.at[pl.ds(row_base + d0, 8)])

    def pick_bias(bias_tab, raw_idx_hbm, out_ref):
      off = (wid % _TPK) * _BPT
      pltpu.sync_copy(raw_idx_hbm.at[pl.ds(off, _BPT)], bidx_v)
      pltpu.async_copy(bias_tab.at[bidx_v], bseg_v, sem).wait()
      pltpu.sync_copy(bseg_v, out_ref.at[pl.ds(off, _BPT)])

    @pl.when(kind == 0)
    def _():
      pltpu.sync_copy(us_hbm, idx_v)
      pltpu.sync_copy(uo_hbm, ord_v)
      pltpu.sync_copy(starts_hbm.at[pl.ds(0, 32)], starts_v)
      pick_rows(ueT, 0, 0, 0, _WBASE)
      pick_bias(bias_u1, uid_hbm, bu_o)

    @pl.when(kind == 1)
    def _():
      pltpu.sync_copy(us_hbm, idx_v)
      pltpu.sync_copy(uo_hbm, ord_v)
      pltpu.sync_copy(starts_hbm.at[pl.ds(0, 32)], starts_v)
      pick_rows(weT, 0, 0, _DIM, _WBASE)

    @pl.when(kind == 2)
    def _():
      pltpu.sync_copy(is_hbm, idx_v)
      pltpu.sync_copy(io_hbm, ord_v)
      pltpu.sync_copy(starts_hbm.at[pl.ds(64, 32)], starts_v)
      pick_rows(ieT, 0, 0, 2 * _DIM, _WBASE)
      pick_bias(bias_i1, iid_hbm, bi_o)

    @pl.when(kind == 3)
    def _():
      pltpu.sync_copy(is_hbm, idx_v)
      pltpu.sync_copy(io_hbm, ord_v)
      pltpu.sync_copy(starts_hbm.at[pl.ds(96, 32)], starts_v)
      pick_rows(weT, _WBASE, _NU, 3 * _DIM, 99936)

  return _gather_sc


def _fm_tc(xt_ref, v_ref, w_ref, bu, bi, bias_ref, out_ref):
  xt = xt_ref[...]
  v = v_ref[...]
  dn = (((0,), (0,)), ((), ()))
  yt = lax.dot_general(v, xt, dn, preferred_element_type=jnp.float32)
  t1 = jnp.sum(yt * yt, axis=0)
  sv = jnp.sum(v * v, axis=1, keepdims=True)
  t2 = lax.dot_general(sv, xt * xt, dn,
                       preferred_element_type=jnp.float32)[0]
  lin = lax.dot_general(w_ref[...], xt, dn,
                        preferred_element_type=jnp.float32)[0]
  out_ref[...] = lin + 0.5 * (t1 - t2) + bu[...] + bi[...] + bias_ref[0]


_fm_call = pl.pallas_call(
    _fm_tc,
    in_specs=[
        pl.BlockSpec(memory_space=pltpu.VMEM),
        pl.BlockSpec(memory_space=pltpu.VMEM),
        pl.BlockSpec(memory_space=pltpu.VMEM),
        pl.BlockSpec(memory_space=pltpu.VMEM),
        pl.BlockSpec(memory_space=pltpu.VMEM),
        pl.BlockSpec(memory_space=pltpu.SMEM),
    ],
    out_shape=jax.ShapeDtypeStruct((_B,), jnp.float32),
)


def _starts(sorted_idx, edges):
  e = jnp.asarray(edges, dtype=jnp.int32)
  s = jnp.sum(sorted_idx[None, :] < e[:, None], axis=1).astype(jnp.int32)
  return jnp.pad(s, (0, 32 - len(edges)))


def kernel(uid_batch, iid_batch, u_nodes, u_adj_ind, u_adj_tp,
           i_nodes, i_adj_ind, i_adj_tp,
           user_emb, item_emb, word_emb, W_lin, V, bias_u, bias_i, bias):
  uid = uid_batch.astype(jnp.int32)
  iid = iid_batch.astype(jnp.int32)
  pos = jnp.arange(_B, dtype=jnp.int32)
  us, uo = lax.sort_key_val(uid, pos)
  isrt, io = lax.sort_key_val(iid, pos)
  su = _starts(us, _EDGES_LO)
  si = _starts(isrt, _EDGES_LO)
  sw = _starts(isrt, _EDGES_HI)
  starts = jnp.concatenate([su, su, si, sw])
  ueT, ieT, weT = user_emb.T, item_emb.T, word_emb.T
  tails = jnp.concatenate([
      jnp.pad(ueT[:, _WBASE:_NU], ((0, 0), (0, 96))),
      jnp.pad(weT[:, _WBASE:_NU], ((0, 0), (0, 96))),
      jnp.pad(ieT[:, _WBASE:_NU], ((0, 0), (0, 96))),
      jnp.pad(weT[:, 199936:2 * _NU], ((0, 0), (0, 64))),
  ])
  xt, bu, bi = _make_gather_sc()(
      uid, iid, us, uo, isrt, io, starts,
      ueT, ieT, weT, tails, bias_u, bias_i)
  return _fm_call(xt, V, W_lin, bu, bi, bias)


# speedup vs baseline: 2.2090x; 1.3390x over previous
import functools

import jax
import jax.numpy as jnp
from jax import lax
from jax.experimental import pallas as pl
from jax.experimental.pallas import tpu as pltpu
from jax.experimental.pallas import tpu_sc as plsc

_DIM = 64
_B = 4096
_NC, _NS, _L = 2, 16, 16
_NW = _NC * _NS
_NU = 100000

_CW = 5120
_NCHUNK = 20
_LASTW = 2688
_WBASE = 99968
_TAILW = 128

_TPK = 8
_BPT = _B // _TPK

_EDGES_LO = [c * _CW for c in range(_NCHUNK)] + [_WBASE, 1 << 30]
_EDGES_HI = [c * _CW - 32 for c in range(_NCHUNK)] + [99936, 1 << 30]


@functools.cache
def _make_gather_sc():
  mesh = plsc.VectorSubcoreMesh(
      core_axis_name="c", subcore_axis_name="s",
      num_cores=_NC, num_subcores=_NS)

  @functools.partial(
      pl.kernel,
      mesh=mesh,
      compiler_params=pltpu.CompilerParams(use_tc_tiling_on_sc=True, needs_layout_passes=False),
      out_type=(
          jax.ShapeDtypeStruct((4 * _DIM, _B), jnp.float32),
          jax.ShapeDtypeStruct((_B,), jnp.float32),
          jax.ShapeDtypeStruct((_B,), jnp.float32),
      ),
      scratch_types=[
          pltpu.VMEM((_B,), jnp.int32),
          pltpu.VMEM((_B,), jnp.int32),
          pltpu.VMEM((32,), jnp.int32),
          pltpu.VMEM((8, _CW), jnp.float32),
          pltpu.VMEM((8, _CW), jnp.float32),
          pltpu.VMEM((8, _B), jnp.float32),
          pltpu.VMEM((_BPT,), jnp.int32),
          pltpu.VMEM((_BPT,), jnp.float32),
          pltpu.SemaphoreType.DMA,
          pltpu.SemaphoreType.DMA,
      ],
  )
  def _gather_sc(uid_hbm, iid_hbm, us_hbm, uo_hbm, is_hbm, io_hbm, starts_hbm,
                 ueT, ieT, weT, tails, bias_u1, bias_i1,
                 xT_o, bu_o, bi_o,
                 idx_v, ord_v, starts_v, buf_a, buf_b, rows_v, bidx_v, bseg_v,
                 sem_a, sem_b):
    wid = lax.axis_index("s") * _NC + lax.axis_index("c")
    kind = wid // _TPK
    d0 = pl.multiple_of((wid % _TPK) * 8, 8)
    lanes = jnp.arange(_L, dtype=jnp.int32)

    def scan_chunk(buf_v, lo, hi, gb, ln):
      zeros = jnp.zeros((_L,), dtype=jnp.int32)

      def urow(u, _):
        usplat = zeros + u

        def body(k, _):
          sl = pl.ds(k * _L, _L)
          pos = k * _L + lanes
          m = (pos >= lo) & (pos < hi)
          iv = idx_v[sl]
          ov = ord_v[sl]
          ic = jnp.clip(iv - gb, 0, ln - 1)
          vals = plsc.load_gather(buf_v, [usplat, ic])
          plsc.store_scatter(rows_v, [usplat, ov], vals, mask=m)
          return 0

        lax.fori_loop(lo // _L, (hi + _L - 1) // _L, body, 0)
        return 0

      lax.fori_loop(0, 8, urow, 0)

    def pick_rows(table, tab_base, idx_base, row_base, gb_tail):
      sv0 = starts_v[pl.ds(0, _L)]
      sv1 = starts_v[pl.ds(_L, _L)]

      def edge(c):
        return sv0[c] if c < _L else sv1[c - _L]

      bufs = (buf_a, buf_b)
      sems = (sem_a, sem_b)
      wins = []
      for c in range(_NCHUNK):
        ln = _CW if c < _NCHUNK - 1 else _LASTW
        n0 = tab_base + c * _CW
        wins.append((table, pl.ds(d0, 8), pl.ds(n0, ln), n0 - idx_base, ln))
      wins.append((tails, pl.ds(row_base + d0, 8), None, gb_tail, _TAILW))

      def start(c):
        tab, sl0, sl1, _, ln = wins[c]
        src = tab.at[sl0] if sl1 is None else tab.at[sl0, sl1]
        return pltpu.async_copy(src, bufs[c % 2].at[:, pl.ds(0, ln)],
                                sems[c % 2])

      pend = start(0)
      for c in range(_NCHUNK + 1):
        nxt = start(c + 1) if c + 1 < _NCHUNK + 1 else None
        pend.wait()
        scan_chunk(bufs[c % 2], edge(c), edge(c + 1), wins[c][3], wins[c][4])
        pend = nxt
      pltpu.sync_copy(rows_v, xT_o.at[pl.ds(row_base + d0, 8)])

    def pick_bias(bias_tab, raw_idx_hbm, out_ref):
      off = (wid % _TPK) * _BPT
      pltpu.sync_copy(raw_idx_hbm.at[pl.ds(off, _BPT)], bidx_v)
      pltpu.async_copy(bias_tab.at[bidx_v], bseg_v, sem_a).wait()
      pltpu.sync_copy(bseg_v, out_ref.at[pl.ds(off, _BPT)])

    @pl.when(kind == 0)
    def _():
      pltpu.sync_copy(us_hbm, idx_v)
      pltpu.sync_copy(uo_hbm, ord_v)
      pltpu.sync_copy(starts_hbm.at[pl.ds(0, 32)], starts_v)
      pick_rows(ueT, 0, 0, 0, _WBASE)
      pick_bias(bias_u1, uid_hbm, bu_o)

    @pl.when(kind == 1)
    def _():
      pltpu.sync_copy(us_hbm, idx_v)
      pltpu.sync_copy(uo_hbm, ord_v)
      pltpu.sync_copy(starts_hbm.at[pl.ds(0, 32)], starts_v)
      pick_rows(weT, 0, 0, _DIM, _WBASE)

    @pl.when(kind == 2)
    def _():
      pltpu.sync_copy(is_hbm, idx_v)
      pltpu.sync_copy(io_hbm, ord_v)
      pltpu.sync_copy(starts_hbm.at[pl.ds(64, 32)], starts_v)
      pick_rows(ieT, 0, 0, 2 * _DIM, _WBASE)
      pick_bias(bias_i1, iid_hbm, bi_o)

    @pl.when(kind == 3)
    def _():
      pltpu.sync_copy(is_hbm, idx_v)
      pltpu.sync_copy(io_hbm, ord_v)
      pltpu.sync_copy(starts_hbm.at[pl.ds(96, 32)], starts_v)
      pick_rows(weT, _WBASE, _NU, 3 * _DIM, 99936)

  return _gather_sc


def _fm_tc(xt_ref, v_ref, w_ref, bu, bi, bias_ref, out_ref):
  xt = xt_ref[...]
  v = v_ref[...]
  dn = (((0,), (0,)), ((), ()))
  yt = lax.dot_general(v, xt, dn, preferred_element_type=jnp.float32)
  t1 = jnp.sum(yt * yt, axis=0)
  sv = jnp.sum(v * v, axis=1, keepdims=True)
  t2 = lax.dot_general(sv, xt * xt, dn,
                       preferred_element_type=jnp.float32)[0]
  lin = lax.dot_general(w_ref[...], xt, dn,
                        preferred_element_type=jnp.float32)[0]
  out_ref[...] = lin + 0.5 * (t1 - t2) + bu[...] + bi[...] + bias_ref[0]


_fm_call = pl.pallas_call(
    _fm_tc,
    in_specs=[
        pl.BlockSpec(memory_space=pltpu.VMEM),
        pl.BlockSpec(memory_space=pltpu.VMEM),
        pl.BlockSpec(memory_space=pltpu.VMEM),
        pl.BlockSpec(memory_space=pltpu.VMEM),
        pl.BlockSpec(memory_space=pltpu.VMEM),
        pl.BlockSpec(memory_space=pltpu.SMEM),
    ],
    out_shape=jax.ShapeDtypeStruct((_B,), jnp.float32),
)


def _starts(sorted_idx, edges):
  e = jnp.asarray(edges, dtype=jnp.int32)
  s = jnp.sum(sorted_idx[None, :] < e[:, None], axis=1).astype(jnp.int32)
  return jnp.pad(s, (0, 32 - len(edges)))


def kernel(uid_batch, iid_batch, u_nodes, u_adj_ind, u_adj_tp,
           i_nodes, i_adj_ind, i_adj_tp,
           user_emb, item_emb, word_emb, W_lin, V, bias_u, bias_i, bias):
  uid = uid_batch.astype(jnp.int32)
  iid = iid_batch.astype(jnp.int32)
  pos = jnp.broadcast_to(jnp.arange(_B, dtype=jnp.int32), (2, _B))
  sk, sv = lax.sort_key_val(jnp.stack([uid, iid]), pos)
  us, isrt = sk[0], sk[1]
  uo, io = sv[0], sv[1]
  su = _starts(us, _EDGES_LO)
  si = _starts(isrt, _EDGES_LO)
  sw = _starts(isrt, _EDGES_HI)
  starts = jnp.concatenate([su, su, si, sw])
  ueT, ieT, weT = user_emb.T, item_emb.T, word_emb.T
  tails = jnp.concatenate([
      jnp.pad(ueT[:, _WBASE:_NU], ((0, 0), (0, 96))),
      jnp.pad(weT[:, _WBASE:_NU], ((0, 0), (0, 96))),
      jnp.pad(ieT[:, _WBASE:_NU], ((0, 0), (0, 96))),
      jnp.pad(weT[:, 199936:2 * _NU], ((0, 0), (0, 64))),
  ])
  xt, bu, bi = _make_gather_sc()(
      uid, iid, us, uo, isrt, io, starts,
      ueT, ieT, weT, tails, bias_u, bias_i)
  return _fm_call(xt, V, W_lin, bu, bi, bias)
